# baseline (speedup 1.0000x reference)
import functools

import numpy as np
import jax
import jax.numpy as jnp
from jax.experimental import pallas as pl

NUM_NODES = 64
OBS_DIM = 1024
GNN_OBS = OBS_DIM // NUM_NODES
GNN_ACT = 2
HIDDEN = 128
LOG_STD_MIN, LOG_STD_MAX = -5.0, 2.0

BATCH_BLOCK = 1024
CHUNK_B = 16
KGRP = 8
GSIZE = CHUNK_B * KGRP
CROWS = CHUNK_B * NUM_NODES


THIRD = 1.0 / 3.0
S6 = float(1.0 / np.sqrt(6.0))


def _edge_coeffs():
    j = jax.lax.rem(jax.lax.broadcasted_iota(jnp.int32, (GSIZE, 1), 0),
                    KGRP)
    j0 = j == 0
    j7 = j == KGRP - 1
    one = jnp.float32(1.0)
    sel = lambda m, v: jnp.where(m, jnp.float32(3.0 * v), one)
    av0 = sel(j0, 0.5)
    lv0 = sel(j0, 0.0)
    uv0 = sel(j0, S6)
    lv1 = sel(j0, S6)
    uv6 = sel(j7, S6)
    av7 = sel(j7, 0.5)
    lv7 = sel(j7, S6)
    uv7 = sel(j7, 0.0)
    return av0, lv0, uv0, lv1, uv6, av7, lv7, uv7


def _fused_kernel(x_ref, w0a_ref, b0_ref, w1_ref, b1_ref, w2s_ref, b2s_ref,
                  mu_ref, std_ref):
    nb = x_ref.shape[0]
    av0, lv0, uv0, lv1, uv6, av7, lv7, uv7 = _edge_coeffs()
    def agg(g):
        prev = [jnp.roll(g[KGRP - 1], 1, axis=0)] + g[:KGRP - 1]
        nxt = g[1:] + [jnp.roll(g[0], -1, axis=0)]
        return [
            av0 * g[0] + lv0 * prev[0] + uv0 * nxt[0],
            (g[1] + nxt[1]) + lv1 * prev[1],
            (g[2] + prev[2]) + nxt[2],
            (g[3] + prev[3]) + nxt[3],
            (g[4] + prev[4]) + nxt[4],
            (g[5] + prev[5]) + nxt[5],
            (g[6] + prev[6]) + uv6 * nxt[6],
            av7 * g[7] + lv7 * prev[7] + uv7 * nxt[7],
        ]

    def body(c, carry):
        b0 = b0_ref[...]
        b1 = b1_ref[...]
        b2s = b2s_ref[...]
        xc = x_ref[pl.ds(c * CHUNK_B, CHUNK_B), :, :].reshape(GSIZE, HIDDEN)
        y = jnp.dot(xc, w0a_ref[...], preferred_element_type=jnp.float32)
        h = [y[:, k * HIDDEN:(k + 1) * HIDDEN] for k in range(KGRP)]
        h = [jax.nn.relu(t + b0) for t in agg(h)]
        w1 = w1_ref[...]
        h = [jnp.dot(t, w1, preferred_element_type=jnp.float32) for t in h]
        h = [jax.nn.relu(t + b1) for t in agg(h)]
        g = agg(h)
        gw = jnp.concatenate(g, axis=1)
        p = jnp.dot(gw, w2s_ref[...], preferred_element_type=jnp.float32) + b2s
        base = c * GSIZE
        ow = KGRP * GNN_ACT
        mu_ref[pl.ds(base, GSIZE), :] = p[:, :ow]
        ls = jnp.tanh(p[:, ow:])
        ls = LOG_STD_MIN + 0.5 * (LOG_STD_MAX - LOG_STD_MIN) * (ls + 1.0)
        std_ref[pl.ds(base, GSIZE), :] = jnp.exp(ls)
        return carry

    jax.lax.fori_loop(0, nb // CHUNK_B, body, 0, unroll=2)


@functools.partial(jax.jit, static_argnames=())
def kernel(obs, W0, b0, W1, b1, W2, b2):
    bs = obs.shape[0]
    out_w = NUM_NODES * GNN_ACT
    grid = (bs // BATCH_BLOCK,)

    x = obs.reshape(bs, KGRP, HIDDEN)
    eye = jnp.eye(KGRP, dtype=jnp.float32)
    third = jnp.float32(THIRD)
    W0all = jnp.kron(eye, W0) * third
    W2s = jnp.concatenate(
        [jnp.kron(eye, W2[:, :GNN_ACT]), jnp.kron(eye, W2[:, GNN_ACT:])],
        axis=1) * third
    b2s = jnp.concatenate(
        [jnp.tile(b2[:GNN_ACT], KGRP), jnp.tile(b2[GNN_ACT:], KGRP)])

    ow = KGRP * GNN_ACT
    mu, std = pl.pallas_call(
        _fused_kernel,
        grid=grid,
        in_specs=[
            pl.BlockSpec((BATCH_BLOCK, KGRP, HIDDEN), lambda i: (i, 0, 0)),
            pl.BlockSpec((HIDDEN, KGRP * HIDDEN), lambda i: (0, 0)),
            pl.BlockSpec((1, HIDDEN), lambda i: (0, 0)),
            pl.BlockSpec((HIDDEN, HIDDEN), lambda i: (0, 0)),
            pl.BlockSpec((1, HIDDEN), lambda i: (0, 0)),
            pl.BlockSpec((KGRP * HIDDEN, 2 * KGRP * GNN_ACT), lambda i: (0, 0)),
            pl.BlockSpec((1, 2 * KGRP * GNN_ACT), lambda i: (0, 0)),
        ],
        out_specs=[
            pl.BlockSpec((BATCH_BLOCK * KGRP, ow), lambda i: (i, 0)),
            pl.BlockSpec((BATCH_BLOCK * KGRP, ow), lambda i: (i, 0)),
        ],
        out_shape=[
            jax.ShapeDtypeStruct((bs * KGRP, ow), jnp.float32),
            jax.ShapeDtypeStruct((bs * KGRP, ow), jnp.float32),
        ],
    )(x, W0all, b0.reshape(1, HIDDEN), W1 * third, b1.reshape(1, HIDDEN),
      W2s, b2s.reshape(1, 2 * KGRP * GNN_ACT))

    return (mu.reshape(bs, out_w), std.reshape(bs, out_w))


# speedup vs baseline: 1.2345x; 1.2345x over previous
import functools

import numpy as np
import jax
import jax.numpy as jnp
from jax.experimental import pallas as pl

NUM_NODES = 64
OBS_DIM = 1024
GNN_OBS = OBS_DIM // NUM_NODES
GNN_ACT = 2
HIDDEN = 128
LOG_STD_MIN, LOG_STD_MAX = -5.0, 2.0

BATCH_BLOCK = 512
CHUNK_B = 32
KGRP = 8
GSIZE = CHUNK_B * KGRP
CROWS = CHUNK_B * NUM_NODES


THIRD = 1.0 / 3.0
S6 = float(1.0 / np.sqrt(6.0))


def _edge_coeffs():
    j = jax.lax.rem(jax.lax.broadcasted_iota(jnp.int32, (GSIZE, 1), 0),
                    KGRP)
    j0 = j == 0
    j7 = j == KGRP - 1
    one = jnp.float32(1.0)
    sel = lambda m, v: jnp.where(m, jnp.float32(3.0 * v), one)
    av0 = sel(j0, 0.5)
    lv0 = sel(j0, 0.0)
    uv0 = sel(j0, S6)
    lv1 = sel(j0, S6)
    uv6 = sel(j7, S6)
    av7 = sel(j7, 0.5)
    lv7 = sel(j7, S6)
    uv7 = sel(j7, 0.0)
    return av0, lv0, uv0, lv1, uv6, av7, lv7, uv7


def _fused_kernel(x_ref, w0a_ref, b0_ref, w1_ref, b1_ref, w2s_ref, b2s_ref,
                  mu_ref, std_ref):
    nb = x_ref.shape[0]
    av0, lv0, uv0, lv1, uv6, av7, lv7, uv7 = _edge_coeffs()
    def agg(g):
        prev = [jnp.roll(g[KGRP - 1], 1, axis=0)] + g[:KGRP - 1]
        nxt = g[1:] + [jnp.roll(g[0], -1, axis=0)]
        return [
            av0 * g[0] + lv0 * prev[0] + uv0 * nxt[0],
            (g[1] + nxt[1]) + lv1 * prev[1],
            (g[2] + prev[2]) + nxt[2],
            (g[3] + prev[3]) + nxt[3],
            (g[4] + prev[4]) + nxt[4],
            (g[5] + prev[5]) + nxt[5],
            (g[6] + prev[6]) + uv6 * nxt[6],
            av7 * g[7] + lv7 * prev[7] + uv7 * nxt[7],
        ]

    def body(c, carry):
        b0 = b0_ref[...]
        b1 = b1_ref[...]
        b2s = b2s_ref[...]
        xc = x_ref[pl.ds(c * CHUNK_B, CHUNK_B), :, :].reshape(GSIZE, HIDDEN)
        y = jnp.dot(xc, w0a_ref[...], preferred_element_type=jnp.float32)
        h = [y[:, k * HIDDEN:(k + 1) * HIDDEN] for k in range(KGRP)]
        h = [jax.nn.relu(t + b0) for t in agg(h)]
        w1 = w1_ref[...]
        h = [jnp.dot(t, w1, preferred_element_type=jnp.float32) for t in h]
        h = [jax.nn.relu(t + b1) for t in agg(h)]
        g = agg(h)
        gw = jnp.concatenate(g, axis=1)
        p = jnp.dot(gw, w2s_ref[...], preferred_element_type=jnp.float32) + b2s
        base = c * GSIZE
        ow = KGRP * GNN_ACT
        mu_ref[pl.ds(base, GSIZE), :] = p[:, :ow]
        ls = jnp.tanh(p[:, ow:])
        ls = LOG_STD_MIN + 0.5 * (LOG_STD_MAX - LOG_STD_MIN) * (ls + 1.0)
        std_ref[pl.ds(base, GSIZE), :] = jnp.exp(ls)
        return carry

    jax.lax.fori_loop(0, nb // CHUNK_B, body, 0, unroll=2)


@functools.partial(jax.jit, static_argnames=())
def kernel(obs, W0, b0, W1, b1, W2, b2):
    bs = obs.shape[0]
    out_w = NUM_NODES * GNN_ACT
    grid = (bs // BATCH_BLOCK,)

    x = obs.reshape(bs, KGRP, HIDDEN)
    eye = jnp.eye(KGRP, dtype=jnp.float32)
    third = jnp.float32(THIRD)
    W0all = jnp.kron(eye, W0) * third
    W2s = jnp.concatenate(
        [jnp.kron(eye, W2[:, :GNN_ACT]), jnp.kron(eye, W2[:, GNN_ACT:])],
        axis=1) * third
    b2s = jnp.concatenate(
        [jnp.tile(b2[:GNN_ACT], KGRP), jnp.tile(b2[GNN_ACT:], KGRP)])

    ow = KGRP * GNN_ACT
    mu, std = pl.pallas_call(
        _fused_kernel,
        grid=grid,
        in_specs=[
            pl.BlockSpec((BATCH_BLOCK, KGRP, HIDDEN), lambda i: (i, 0, 0)),
            pl.BlockSpec((HIDDEN, KGRP * HIDDEN), lambda i: (0, 0)),
            pl.BlockSpec((1, HIDDEN), lambda i: (0, 0)),
            pl.BlockSpec((HIDDEN, HIDDEN), lambda i: (0, 0)),
            pl.BlockSpec((1, HIDDEN), lambda i: (0, 0)),
            pl.BlockSpec((KGRP * HIDDEN, 2 * KGRP * GNN_ACT), lambda i: (0, 0)),
            pl.BlockSpec((1, 2 * KGRP * GNN_ACT), lambda i: (0, 0)),
        ],
        out_specs=[
            pl.BlockSpec((BATCH_BLOCK * KGRP, ow), lambda i: (i, 0)),
            pl.BlockSpec((BATCH_BLOCK * KGRP, ow), lambda i: (i, 0)),
        ],
        out_shape=[
            jax.ShapeDtypeStruct((bs * KGRP, ow), jnp.float32),
            jax.ShapeDtypeStruct((bs * KGRP, ow), jnp.float32),
        ],
    )(x, W0all, b0.reshape(1, HIDDEN), W1 * third, b1.reshape(1, HIDDEN),
      W2s, b2s.reshape(1, 2 * KGRP * GNN_ACT))

    return (mu.reshape(bs, out_w), std.reshape(bs, out_w))


# speedup vs baseline: 1.3063x; 1.0582x over previous
import functools

import numpy as np
import jax
import jax.numpy as jnp
from jax.experimental import pallas as pl

NUM_NODES = 64
OBS_DIM = 1024
GNN_OBS = OBS_DIM // NUM_NODES
GNN_ACT = 2
HIDDEN = 128
LOG_STD_MIN, LOG_STD_MAX = -5.0, 2.0

BATCH_BLOCK = 512
CHUNK_B = 64
KGRP = 8
GSIZE = CHUNK_B * KGRP
CROWS = CHUNK_B * NUM_NODES


THIRD = 1.0 / 3.0
S6 = float(1.0 / np.sqrt(6.0))


def _edge_coeffs():
    j = jax.lax.rem(jax.lax.broadcasted_iota(jnp.int32, (GSIZE, 1), 0),
                    KGRP)
    j0 = j == 0
    j7 = j == KGRP - 1
    one = jnp.float32(1.0)
    sel = lambda m, v: jnp.where(m, jnp.float32(3.0 * v), one)
    av0 = sel(j0, 0.5)
    lv0 = sel(j0, 0.0)
    uv0 = sel(j0, S6)
    lv1 = sel(j0, S6)
    uv6 = sel(j7, S6)
    av7 = sel(j7, 0.5)
    lv7 = sel(j7, S6)
    uv7 = sel(j7, 0.0)
    return av0, lv0, uv0, lv1, uv6, av7, lv7, uv7


def _fused_kernel(x_ref, w0a_ref, b0_ref, w1_ref, b1_ref, w2s_ref, b2s_ref,
                  mu_ref, std_ref):
    nb = x_ref.shape[0]
    av0, lv0, uv0, lv1, uv6, av7, lv7, uv7 = _edge_coeffs()
    def agg(g):
        prev = [jnp.roll(g[KGRP - 1], 1, axis=0)] + g[:KGRP - 1]
        nxt = g[1:] + [jnp.roll(g[0], -1, axis=0)]
        return [
            av0 * g[0] + lv0 * prev[0] + uv0 * nxt[0],
            (g[1] + nxt[1]) + lv1 * prev[1],
            (g[2] + prev[2]) + nxt[2],
            (g[3] + prev[3]) + nxt[3],
            (g[4] + prev[4]) + nxt[4],
            (g[5] + prev[5]) + nxt[5],
            (g[6] + prev[6]) + uv6 * nxt[6],
            av7 * g[7] + lv7 * prev[7] + uv7 * nxt[7],
        ]

    def body(c, carry):
        b0 = b0_ref[...]
        b1 = b1_ref[...]
        b2s = b2s_ref[...]
        xc = x_ref[pl.ds(c * CHUNK_B, CHUNK_B), :, :].reshape(GSIZE, HIDDEN)
        y = jnp.dot(xc, w0a_ref[...], preferred_element_type=jnp.float32)
        h = [y[:, k * HIDDEN:(k + 1) * HIDDEN] for k in range(KGRP)]
        h = [jax.nn.relu(t + b0) for t in agg(h)]
        w1 = w1_ref[...]
        h = [jnp.dot(t, w1, preferred_element_type=jnp.float32) for t in h]
        h = [jax.nn.relu(t + b1) for t in agg(h)]
        g = agg(h)
        gw = jnp.concatenate(g, axis=1)
        p = jnp.dot(gw, w2s_ref[...], preferred_element_type=jnp.float32) + b2s
        base = c * GSIZE
        ow = KGRP * GNN_ACT
        mu_ref[pl.ds(base, GSIZE), :] = p[:, :ow]
        ls = jnp.tanh(p[:, ow:])
        ls = LOG_STD_MIN + 0.5 * (LOG_STD_MAX - LOG_STD_MIN) * (ls + 1.0)
        std_ref[pl.ds(base, GSIZE), :] = jnp.exp(ls)
        return carry

    jax.lax.fori_loop(0, nb // CHUNK_B, body, 0, unroll=2)


@functools.partial(jax.jit, static_argnames=())
def kernel(obs, W0, b0, W1, b1, W2, b2):
    bs = obs.shape[0]
    out_w = NUM_NODES * GNN_ACT
    grid = (bs // BATCH_BLOCK,)

    x = obs.reshape(bs, KGRP, HIDDEN)
    eye = jnp.eye(KGRP, dtype=jnp.float32)
    third = jnp.float32(THIRD)
    W0all = jnp.kron(eye, W0) * third
    W2s = jnp.concatenate(
        [jnp.kron(eye, W2[:, :GNN_ACT]), jnp.kron(eye, W2[:, GNN_ACT:])],
        axis=1) * third
    b2s = jnp.concatenate(
        [jnp.tile(b2[:GNN_ACT], KGRP), jnp.tile(b2[GNN_ACT:], KGRP)])

    ow = KGRP * GNN_ACT
    mu, std = pl.pallas_call(
        _fused_kernel,
        grid=grid,
        in_specs=[
            pl.BlockSpec((BATCH_BLOCK, KGRP, HIDDEN), lambda i: (i, 0, 0)),
            pl.BlockSpec((HIDDEN, KGRP * HIDDEN), lambda i: (0, 0)),
            pl.BlockSpec((1, HIDDEN), lambda i: (0, 0)),
            pl.BlockSpec((HIDDEN, HIDDEN), lambda i: (0, 0)),
            pl.BlockSpec((1, HIDDEN), lambda i: (0, 0)),
            pl.BlockSpec((KGRP * HIDDEN, 2 * KGRP * GNN_ACT), lambda i: (0, 0)),
            pl.BlockSpec((1, 2 * KGRP * GNN_ACT), lambda i: (0, 0)),
        ],
        out_specs=[
            pl.BlockSpec((BATCH_BLOCK * KGRP, ow), lambda i: (i, 0)),
            pl.BlockSpec((BATCH_BLOCK * KGRP, ow), lambda i: (i, 0)),
        ],
        out_shape=[
            jax.ShapeDtypeStruct((bs * KGRP, ow), jnp.float32),
            jax.ShapeDtypeStruct((bs * KGRP, ow), jnp.float32),
        ],
    )(x, W0all, b0.reshape(1, HIDDEN), W1 * third, b1.reshape(1, HIDDEN),
      W2s, b2s.reshape(1, 2 * KGRP * GNN_ACT))

    return (mu.reshape(bs, out_w), std.reshape(bs, out_w))


# speedup vs baseline: 1.3744x; 1.0521x over previous
import functools

import numpy as np
import jax
import jax.numpy as jnp
from jax.experimental import pallas as pl

NUM_NODES = 64
OBS_DIM = 1024
GNN_OBS = OBS_DIM // NUM_NODES
GNN_ACT = 2
HIDDEN = 128
LOG_STD_MIN, LOG_STD_MAX = -5.0, 2.0

BATCH_BLOCK = 512
CHUNK_B = 128
KGRP = 8
GSIZE = CHUNK_B * KGRP
CROWS = CHUNK_B * NUM_NODES


THIRD = 1.0 / 3.0
S6 = float(1.0 / np.sqrt(6.0))


def _edge_coeffs():
    j = jax.lax.rem(jax.lax.broadcasted_iota(jnp.int32, (GSIZE, 1), 0),
                    KGRP)
    j0 = j == 0
    j7 = j == KGRP - 1
    one = jnp.float32(1.0)
    sel = lambda m, v: jnp.where(m, jnp.float32(3.0 * v), one)
    av0 = sel(j0, 0.5)
    lv0 = sel(j0, 0.0)
    uv0 = sel(j0, S6)
    lv1 = sel(j0, S6)
    uv6 = sel(j7, S6)
    av7 = sel(j7, 0.5)
    lv7 = sel(j7, S6)
    uv7 = sel(j7, 0.0)
    return av0, lv0, uv0, lv1, uv6, av7, lv7, uv7


def _fused_kernel(x_ref, w0a_ref, b0_ref, w1_ref, b1_ref, w2s_ref, b2s_ref,
                  mu_ref, std_ref):
    nb = x_ref.shape[0]
    av0, lv0, uv0, lv1, uv6, av7, lv7, uv7 = _edge_coeffs()
    def agg(g):
        prev = [jnp.roll(g[KGRP - 1], 1, axis=0)] + g[:KGRP - 1]
        nxt = g[1:] + [jnp.roll(g[0], -1, axis=0)]
        return [
            av0 * g[0] + lv0 * prev[0] + uv0 * nxt[0],
            (g[1] + nxt[1]) + lv1 * prev[1],
            (g[2] + prev[2]) + nxt[2],
            (g[3] + prev[3]) + nxt[3],
            (g[4] + prev[4]) + nxt[4],
            (g[5] + prev[5]) + nxt[5],
            (g[6] + prev[6]) + uv6 * nxt[6],
            av7 * g[7] + lv7 * prev[7] + uv7 * nxt[7],
        ]

    def body(c, carry):
        b0 = b0_ref[...]
        b1 = b1_ref[...]
        b2s = b2s_ref[...]
        xc = x_ref[pl.ds(c * CHUNK_B, CHUNK_B), :, :].reshape(GSIZE, HIDDEN)
        y = jnp.dot(xc, w0a_ref[...], preferred_element_type=jnp.float32)
        h = [y[:, k * HIDDEN:(k + 1) * HIDDEN] for k in range(KGRP)]
        h = [jax.nn.relu(t + b0) for t in agg(h)]
        w1 = w1_ref[...]
        h = [jnp.dot(t, w1, preferred_element_type=jnp.float32) for t in h]
        h = [jax.nn.relu(t + b1) for t in agg(h)]
        g = agg(h)
        gw = jnp.concatenate(g, axis=1)
        p = jnp.dot(gw, w2s_ref[...], preferred_element_type=jnp.float32) + b2s
        base = c * GSIZE
        ow = KGRP * GNN_ACT
        mu_ref[pl.ds(base, GSIZE), :] = p[:, :ow]
        ls = jnp.tanh(p[:, ow:])
        ls = LOG_STD_MIN + 0.5 * (LOG_STD_MAX - LOG_STD_MIN) * (ls + 1.0)
        std_ref[pl.ds(base, GSIZE), :] = jnp.exp(ls)
        return carry

    jax.lax.fori_loop(0, nb // CHUNK_B, body, 0, unroll=2)


@functools.partial(jax.jit, static_argnames=())
def kernel(obs, W0, b0, W1, b1, W2, b2):
    bs = obs.shape[0]
    out_w = NUM_NODES * GNN_ACT
    grid = (bs // BATCH_BLOCK,)

    x = obs.reshape(bs, KGRP, HIDDEN)
    eye = jnp.eye(KGRP, dtype=jnp.float32)
    third = jnp.float32(THIRD)
    W0all = jnp.kron(eye, W0) * third
    W2s = jnp.concatenate(
        [jnp.kron(eye, W2[:, :GNN_ACT]), jnp.kron(eye, W2[:, GNN_ACT:])],
        axis=1) * third
    b2s = jnp.concatenate(
        [jnp.tile(b2[:GNN_ACT], KGRP), jnp.tile(b2[GNN_ACT:], KGRP)])

    ow = KGRP * GNN_ACT
    mu, std = pl.pallas_call(
        _fused_kernel,
        grid=grid,
        in_specs=[
            pl.BlockSpec((BATCH_BLOCK, KGRP, HIDDEN), lambda i: (i, 0, 0)),
            pl.BlockSpec((HIDDEN, KGRP * HIDDEN), lambda i: (0, 0)),
            pl.BlockSpec((1, HIDDEN), lambda i: (0, 0)),
            pl.BlockSpec((HIDDEN, HIDDEN), lambda i: (0, 0)),
            pl.BlockSpec((1, HIDDEN), lambda i: (0, 0)),
            pl.BlockSpec((KGRP * HIDDEN, 2 * KGRP * GNN_ACT), lambda i: (0, 0)),
            pl.BlockSpec((1, 2 * KGRP * GNN_ACT), lambda i: (0, 0)),
        ],
        out_specs=[
            pl.BlockSpec((BATCH_BLOCK * KGRP, ow), lambda i: (i, 0)),
            pl.BlockSpec((BATCH_BLOCK * KGRP, ow), lambda i: (i, 0)),
        ],
        out_shape=[
            jax.ShapeDtypeStruct((bs * KGRP, ow), jnp.float32),
            jax.ShapeDtypeStruct((bs * KGRP, ow), jnp.float32),
        ],
    )(x, W0all, b0.reshape(1, HIDDEN), W1 * third, b1.reshape(1, HIDDEN),
      W2s, b2s.reshape(1, 2 * KGRP * GNN_ACT))

    return (mu.reshape(bs, out_w), std.reshape(bs, out_w))
